# baseline (speedup 1.0000x reference)
import jax
import jax.numpy as jnp
from jax import lax
from jax.experimental import pallas as pl
from jax.experimental.pallas import tpu as pltpu
from jax.experimental.pallas import tpu_sc as plsc

VOCAB = 1000000
EMBED = 32
BATCH = 4096
SEQ = 200

_INFO = plsc.get_sparse_core_info()
_NC = _INFO.num_cores
_NS = _INFO.num_subcores
_L = _INFO.num_lanes
_NW = _NC * _NS
_BPW = BATCH // _NW
_IPW = _BPW * SEQ
_CH = 128
_NCH = _IPW // _CH

_TC_ROWS = 32768


def _tc_body(w_ref, table_ref, t_ref):
    t_ref[...] = jax.lax.dot_general(
        w_ref[...], table_ref[...], (((1,), (1,)), ((), ())),
        preferred_element_type=jnp.float32)[0, :]


def _tc_matvec(table, w_row):
    grid = pl.cdiv(VOCAB, _TC_ROWS)
    return pl.pallas_call(
        _tc_body,
        grid=(grid,),
        in_specs=[
            pl.BlockSpec((1, EMBED), lambda i: (0, 0)),
            pl.BlockSpec((_TC_ROWS, EMBED), lambda i: (i, 0)),
        ],
        out_specs=pl.BlockSpec((_TC_ROWS,), lambda i: (i,)),
        out_shape=jax.ShapeDtypeStruct((VOCAB,), jnp.float32),
    )(w_row, table)


_P1_CHUNK = 384
_P1_NCH = 999936 // _P1_CHUNK
_P1_GMAX = (_P1_NCH + _NW - 1) // _NW
_P1_TAIL = 999936
_T_PAD = 1000064


def _p1_body(w_hbm, table_hbm, t_hbm, w_v, buf0, buf1, tch0, tch1,
             sem0, sem1):
    wid = lax.axis_index("s") * _NC + lax.axis_index("c")
    pltpu.sync_copy(w_hbm, w_v)
    lanes = lax.iota(jnp.int32, _L)
    w0 = w_v[pl.ds(0, _L)]
    w1 = w_v[pl.ds(_L, _L)]
    zero = jnp.zeros((_L,), jnp.float32)

    def dot_rows(buf, tch, nrows):
        def row(r, lanevec):
            v = buf[r, pl.ds(0, _L)] * w0 + buf[r, pl.ds(_L, _L)] * w1
            s = jnp.sum(v)
            lanevec = jnp.where(lanes == r % _L, s, lanevec)

            @pl.when(r % _L == _L - 1)
            def _():
                tch[pl.ds(pl.multiple_of((r // _L) * _L, 8), _L)] = lanevec

            return lanevec

        lax.fori_loop(0, nrows, row, zero)

    def row_slice(g):
        c = wid + _NW * g
        return c, pl.ds(pl.multiple_of(c * _P1_CHUNK, 128), _P1_CHUNK)

    def issue(g, buf, sem):
        c, sl = row_slice(g)

        @pl.when(c < _P1_NCH)
        def _():
            pltpu.async_copy(table_hbm.at[sl], buf, sem)

    def process(g, buf, tch, sem):
        c, sl = row_slice(g)

        @pl.when(c < _P1_NCH)
        def _():
            pltpu.make_async_copy(table_hbm.at[sl], buf, sem).wait()
            dot_rows(buf, tch, _P1_CHUNK)
            pltpu.sync_copy(tch, t_hbm.at[sl])
            issue(g + 2, buf, sem)

    issue(jnp.int32(0), buf0, sem0)
    issue(jnp.int32(1), buf1, sem1)

    def outer(gg, carry):
        process(2 * gg, buf0, tch0, sem0)
        process(2 * gg + 1, buf1, tch1, sem1)
        return carry

    lax.fori_loop(0, (_P1_GMAX + 1) // 2, outer, 0)

    @pl.when(wid == _NW - 1)
    def _():
        pltpu.sync_copy(table_hbm.at[pl.ds(_P1_TAIL, 64)],
                        buf0.at[pl.ds(0, 64)])
        dot_rows(buf0, tch0, 64)
        pltpu.sync_copy(tch0.at[pl.ds(0, 128)],
                        t_hbm.at[pl.ds(_P1_TAIL, 128)])


def _sc_matvec(w_flat, table):
    mesh = plsc.VectorSubcoreMesh(core_axis_name="c", subcore_axis_name="s")
    return pl.kernel(
        _p1_body,
        jax.ShapeDtypeStruct((_T_PAD,), jnp.float32),
        mesh=mesh,
        scratch_types=[
            pltpu.VMEM((EMBED,), jnp.float32),
            pltpu.VMEM((_P1_CHUNK, EMBED), jnp.float32),
            pltpu.VMEM((_P1_CHUNK, EMBED), jnp.float32),
            pltpu.VMEM((_P1_CHUNK,), jnp.float32),
            pltpu.VMEM((_P1_CHUNK,), jnp.float32),
            pltpu.SemaphoreType.DMA,
            pltpu.SemaphoreType.DMA,
        ],
        compiler_params=pltpu.CompilerParams(
            needs_layout_passes=False, use_tc_tiling_on_sc=True),
    )(w_flat, table)


def _sc_body(x_hbm, params_hbm, t_hbm, out_hbm, idx_v, params_v, vals_v,
             outs_v, sem):
    wid = lax.axis_index("s") * _NC + lax.axis_index("c")
    base = wid * _IPW

    pltpu.sync_copy(x_hbm.at[pl.ds(base, _IPW)], idx_v)
    pltpu.sync_copy(params_hbm, params_v)
    bias_v = params_v[pl.ds(0, _L)]

    def fire(c, carry):
        off = pl.multiple_of(c * _CH, 8)
        pltpu.async_copy(
            t_hbm.at[idx_v.at[pl.ds(off, _CH)]], vals_v.at[pl.ds(off, _CH)],
            sem)
        return carry

    lax.fori_loop(0, _NCH, fire, 0)
    pltpu.make_async_copy(t_hbm.at[idx_v], vals_v, sem).wait()

    lanes = lax.iota(jnp.int32, _L)
    nfull = SEQ // _L
    tail_keep = lanes >= (_L - SEQ % _L)
    zero = jnp.zeros((_L,), jnp.float32)

    def elem(e, lanevec):
        off = pl.multiple_of(e * SEQ, 8)
        acc = zero
        for k in range(nfull):
            acc = acc + vals_v[pl.ds(off + k * _L, _L)]
        tail = vals_v[pl.ds(off + SEQ - _L, _L)]
        acc = acc + jnp.where(tail_keep, tail, zero)
        s = jnp.sum(acc)
        lanevec = jnp.where(lanes == e % _L, s, lanevec)

        @pl.when(e % _L == _L - 1)
        def _():
            outs_v[pl.ds(pl.multiple_of((e // _L) * _L, _L), _L)] = lanevec

        return lanevec

    lax.fori_loop(0, _BPW, elem, zero)

    one = jnp.float32(1.0)
    for k in range(_BPW // _L):
        z = outs_v[pl.ds(k * _L, _L)] + bias_v
        outs_v[pl.ds(k * _L, _L)] = one / (one + jnp.exp(-z))

    pltpu.sync_copy(outs_v, out_hbm.at[pl.ds(wid * _BPW, _BPW)])


def _sc_gather(x_flat, params, t):
    mesh = plsc.VectorSubcoreMesh(core_axis_name="c", subcore_axis_name="s")
    return pl.kernel(
        _sc_body,
        jax.ShapeDtypeStruct((BATCH,), jnp.float32),
        mesh=mesh,
        scratch_types=[
            pltpu.VMEM((_IPW,), jnp.int32),
            pltpu.VMEM((_L,), jnp.float32),
            pltpu.VMEM((_IPW,), jnp.float32),
            pltpu.VMEM((_BPW,), jnp.float32),
            pltpu.SemaphoreType.DMA,
        ],
        compiler_params=pltpu.CompilerParams(
            needs_layout_passes=False, use_tc_tiling_on_sc=False),
    )(x_flat, params, t)


@jax.jit
def _run(x, emb_table, fc_w, fc_b):
    x_flat = x.reshape(-1).astype(jnp.int32)
    w_flat = fc_w.reshape(EMBED) * jnp.float32(1.0 / SEQ)
    params = jnp.broadcast_to(fc_b.reshape(-1), (_L,))
    t = _sc_matvec(w_flat, emb_table)
    return _sc_gather(x_flat, params, t)


def kernel(x, emb_table, fc_w, fc_b):
    return _run(x, emb_table, fc_w, fc_b).reshape(BATCH, 1)
